# Initial kernel scaffold; baseline (speedup 1.0000x reference)
import jax
import jax.numpy as jnp
from jax.experimental import pallas as pl


def kernel(x, edge_index, batch, W1, b1, W2, b2, W3, b3, Wl1, bl1, Wl2, bl2):
    raise NotImplementedError("write your pallas kernel here")


# speedup vs baseline: 10.9972x; 10.9972x over previous
import functools

import jax
import jax.numpy as jnp
from jax import lax
from jax.experimental import pallas as pl
from jax.experimental.pallas import tpu as pltpu
from jax.experimental.pallas import tpu_sc as plsc

N_NODES = 50000
HALF = 25000
ACC_ROWS = 25088
N_EDGES = 800000
HID = 64
NUM_GRAPHS = 64

NSUB = 16
NCORE = 2
CHUNK = 1024
CH_ROWS = CHUNK // 128
N_CHUNKS = 49
TILE_EDGES = CHUNK * N_CHUNKS
EPAD = TILE_EDGES * NSUB
EP_ROWS = EPAD // 128
TILE_ROWS = TILE_EDGES // 128

ROW_BLK = 2000
N_BLKS = N_NODES // ROW_BLK


def _mesh():
    return plsc.VectorSubcoreMesh(
        core_axis_name="c", subcore_axis_name="s",
        num_cores=NCORE, num_subcores=NSUB)


def _prep_body(dst_hbm, cnt16_hbm, dstl_hbm, dstb, dstlb, ones_v, zbuf,
               accd, sem):
    c = lax.axis_index("c")
    s = lax.axis_index("s")

    def fill(i, _):
        zbuf[i, :] = jnp.zeros((16,), jnp.float32)
        return _
    lax.fori_loop(0, 1568, fill, None)

    def fill1(i, _):
        ones_v[i, :] = jnp.ones((16,), jnp.float32)
        return _
    lax.fori_loop(0, 128, fill1, None)

    pltpu.sync_copy(zbuf, accd.at[pl.ds(s * 1568, 1568)])
    plsc.subcore_barrier()

    def chunk(k, _):
        rb = s * TILE_ROWS + k * CH_ROWS
        pltpu.sync_copy(dst_hbm.at[pl.ds(rb, CH_ROWS)], dstb)
        for j in range(CH_ROWS):
            for v in range(8):
                d16 = dstb[j, pl.ds(v * 16, 16)]
                loc = d16 - c * HALF
                ok = (loc >= 0) & (loc < HALF)
                dstlb[j, pl.ds(v * 16, 16)] = jnp.where(ok, loc, HALF)
        pltpu.sync_copy(dstlb, dstl_hbm.at[c, pl.ds(rb, CH_ROWS)])

        def sub(j, _):
            pltpu.sync_copy(ones_v, accd.at[dstlb.at[j]], add=True)
            return _
        lax.fori_loop(0, CH_ROWS, sub, None)
        return _
    lax.fori_loop(0, N_CHUNKS, chunk, None)

    plsc.subcore_barrier()
    pltpu.sync_copy(accd.at[pl.ds(s * 1560, 1560)],
                    cnt16_hbm.at[pl.ds(c * HALF + s * 1560, 1560)])

    @pl.when(s == 0)
    def _():
        pltpu.sync_copy(accd.at[pl.ds(24960, 40)],
                        cnt16_hbm.at[pl.ds(c * HALF + 24960, 40)])


def _sc_prep(dst_p):
    fn = pl.kernel(
        _prep_body,
        out_type=(jax.ShapeDtypeStruct((N_NODES, 16), jnp.float32),
                  jax.ShapeDtypeStruct((NCORE, EP_ROWS, 128), jnp.int32)),
        mesh=_mesh(),
        compiler_params=pltpu.CompilerParams(use_tc_tiling_on_sc=False),
        scratch_types=[
            pltpu.VMEM((CH_ROWS, 128), jnp.int32),
            pltpu.VMEM((CH_ROWS, 128), jnp.int32),
            pltpu.VMEM((128, 16), jnp.float32),
            pltpu.VMEM((1568, 16), jnp.float32),
            pltpu.VMEM_SHARED((ACC_ROWS, 16), jnp.float32),
            pltpu.SemaphoreType.DMA,
        ],
    )
    return fn(dst_p)


def _agg_body(hs_hbm, src_hbm, dstl_hbm, agg_hbm, srcb, dstlb, rows, acc,
              sem):
    c = lax.axis_index("c")
    s = lax.axis_index("s")

    pltpu.sync_copy(hs_hbm.at[pl.ds(c * HALF + s * 1560, 1560)],
                    acc.at[pl.ds(s * 1560, 1560)])

    @pl.when(s == 0)
    def _():
        pltpu.sync_copy(hs_hbm.at[pl.ds(c * HALF + 24960, 40)],
                        acc.at[pl.ds(24960, 40)])
    plsc.subcore_barrier()

    def chunk(k, _):
        rb = s * TILE_ROWS + k * CH_ROWS
        pltpu.sync_copy(src_hbm.at[pl.ds(rb, CH_ROWS)], srcb)
        pltpu.sync_copy(dstl_hbm.at[c, pl.ds(rb, CH_ROWS)], dstlb)

        def sub(j, _):
            pltpu.async_copy(hs_hbm.at[srcb.at[j]], rows, sem).wait()
            pltpu.sync_copy(rows, acc.at[dstlb.at[j]], add=True)
            return _
        lax.fori_loop(0, CH_ROWS, sub, None)
        return _
    lax.fori_loop(0, N_CHUNKS, chunk, None)

    plsc.subcore_barrier()
    pltpu.sync_copy(acc.at[pl.ds(s * 1560, 1560)],
                    agg_hbm.at[pl.ds(c * HALF + s * 1560, 1560)])

    @pl.when(s == 0)
    def _():
        pltpu.sync_copy(acc.at[pl.ds(24960, 40)],
                        agg_hbm.at[pl.ds(c * HALF + 24960, 40)])


def _sc_agg(hs, src_p, dstl):
    fn = pl.kernel(
        _agg_body,
        out_type=jax.ShapeDtypeStruct((N_NODES, HID), jnp.float32),
        mesh=_mesh(),
        compiler_params=pltpu.CompilerParams(use_tc_tiling_on_sc=False),
        scratch_types=[
            pltpu.VMEM((CH_ROWS, 128), jnp.int32),
            pltpu.VMEM((CH_ROWS, 128), jnp.int32),
            pltpu.VMEM((128, HID), jnp.float32),
            pltpu.VMEM_SHARED((ACC_ROWS, HID), jnp.float32),
            pltpu.SemaphoreType.DMA,
        ],
    )
    return fn(hs, src_p, dstl)


def _bcast64(cnt16):
    dinv16 = lax.rsqrt(cnt16 + 1.0)
    sel = (lax.broadcasted_iota(jnp.int32, (16, HID), 0) == 0
           ).astype(jnp.float32)
    return lax.dot_general(dinv16, sel, (((1,), (0,)), ((), ())),
                           preferred_element_type=jnp.float32)


def _tc1_body(x_ref, w_ref, cnt_ref, out_ref):
    dinv = _bcast64(cnt_ref[...])
    h = lax.dot_general(x_ref[...], w_ref[...], (((1,), (1,)), ((), ())),
                        preferred_element_type=jnp.float32)
    out_ref[...] = h * dinv


def _tc_layer1(x, W1, cnt16):
    return pl.pallas_call(
        _tc1_body,
        grid=(N_BLKS,),
        in_specs=[
            pl.BlockSpec((ROW_BLK, 20), lambda i: (i, 0)),
            pl.BlockSpec((HID, 20), lambda i: (0, 0)),
            pl.BlockSpec((ROW_BLK, 16), lambda i: (i, 0)),
        ],
        out_specs=pl.BlockSpec((ROW_BLK, HID), lambda i: (i, 0)),
        out_shape=jax.ShapeDtypeStruct((N_NODES, HID), jnp.float32),
    )(x, W1, cnt16)


def _tc2_body(agg_ref, w_ref, b_ref, cnt_ref, out_ref):
    dinv = _bcast64(cnt_ref[...])
    h = jnp.maximum(agg_ref[...] * dinv + b_ref[0:1, :], 0.0)
    hw = lax.dot_general(h, w_ref[...], (((1,), (1,)), ((), ())),
                         preferred_element_type=jnp.float32)
    out_ref[...] = hw * dinv


def _tc_layer(agg, W, b8, cnt16):
    return pl.pallas_call(
        _tc2_body,
        grid=(N_BLKS,),
        in_specs=[
            pl.BlockSpec((ROW_BLK, HID), lambda i: (i, 0)),
            pl.BlockSpec((HID, HID), lambda i: (0, 0)),
            pl.BlockSpec((8, HID), lambda i: (0, 0)),
            pl.BlockSpec((ROW_BLK, 16), lambda i: (i, 0)),
        ],
        out_specs=pl.BlockSpec((ROW_BLK, HID), lambda i: (i, 0)),
        out_shape=jax.ShapeDtypeStruct((N_NODES, HID), jnp.float32),
    )(agg, W, b8, cnt16)


def _tc3_body(agg_ref, b_ref, cnt_ref, batch_ref, wl1_ref, bl1_ref,
              wl2_ref, bl2_ref, out_ref, pool_acc):
    i = pl.program_id(0)

    @pl.when(i == 0)
    def _():
        pool_acc[...] = jnp.zeros((NUM_GRAPHS, 128), jnp.float32)

    dinv = _bcast64(cnt_ref[...])
    h3 = jnp.maximum(agg_ref[...] * dinv + b_ref[0:1, :], 0.0)
    bids = batch_ref[0, 0, :]
    oh = (lax.broadcasted_iota(jnp.int32, (NUM_GRAPHS, ROW_BLK), 0)
          == bids[None, :]).astype(jnp.float32)
    psum = lax.dot_general(oh, h3, (((1,), (0,)), ((), ())),
                           preferred_element_type=jnp.float32)
    cnt64 = lax.dot_general(oh, jnp.ones((ROW_BLK, HID), jnp.float32),
                            (((1,), (0,)), ((), ())),
                            preferred_element_type=jnp.float32)
    pool_acc[:, 0:HID] += psum
    pool_acc[:, HID:128] += cnt64

    @pl.when(i == N_BLKS - 1)
    def _():
        pooled = pool_acc[:, 0:HID] / jnp.maximum(pool_acc[:, HID:128], 1.0)
        h = jnp.maximum(
            lax.dot_general(pooled, wl1_ref[...], (((1,), (1,)), ((), ())),
                            preferred_element_type=jnp.float32)
            + bl1_ref[0:1, :], 0.0)
        o = lax.dot_general(h, wl2_ref[...], (((1,), (1,)), ((), ())),
                            preferred_element_type=jnp.float32)
        out_ref[...] = o + bl2_ref[0:1, :]


def _tc_final(agg3, b38, cnt16, batch3, Wl1, bl18, Wl28, bl28):
    return pl.pallas_call(
        _tc3_body,
        grid=(N_BLKS,),
        in_specs=[
            pl.BlockSpec((ROW_BLK, HID), lambda i: (i, 0)),
            pl.BlockSpec((8, HID), lambda i: (0, 0)),
            pl.BlockSpec((ROW_BLK, 16), lambda i: (i, 0)),
            pl.BlockSpec((1, 1, ROW_BLK), lambda i: (i, 0, 0)),
            pl.BlockSpec((HID, HID), lambda i: (0, 0)),
            pl.BlockSpec((8, HID), lambda i: (0, 0)),
            pl.BlockSpec((128, HID), lambda i: (0, 0)),
            pl.BlockSpec((8, 128), lambda i: (0, 0)),
        ],
        out_specs=pl.BlockSpec((NUM_GRAPHS, 128), lambda i: (0, 0)),
        out_shape=jax.ShapeDtypeStruct((NUM_GRAPHS, 128), jnp.float32),
        scratch_shapes=[pltpu.VMEM((NUM_GRAPHS, 128), jnp.float32)],
    )(agg3, b38, cnt16, batch3, Wl1, bl18, Wl28, bl28)


@jax.jit
def kernel(x, edge_index, batch, W1, b1, W2, b2, W3, b3, Wl1, bl1, Wl2, bl2):
    src = edge_index[0].astype(jnp.int32)
    dst = edge_index[1].astype(jnp.int32)
    npad = EPAD - N_EDGES
    src_p = jnp.concatenate(
        [src, jnp.zeros((npad,), jnp.int32)]).reshape(EP_ROWS, 128)
    dst_p = jnp.concatenate(
        [dst, jnp.full((npad,), N_NODES, jnp.int32)]).reshape(EP_ROWS, 128)
    batch3 = batch.astype(jnp.int32).reshape(N_BLKS, 1, ROW_BLK)

    b18 = jnp.broadcast_to(b1[None, :], (8, HID))
    b28 = jnp.broadcast_to(b2[None, :], (8, HID))
    b38 = jnp.broadcast_to(b3[None, :], (8, HID))
    bl18 = jnp.broadcast_to(bl1[None, :], (8, HID))
    Wl2p = jnp.zeros((128, HID), jnp.float32).at[0:1, :].set(Wl2)
    bl28 = jnp.broadcast_to(bl2[None, :], (8, 128))

    cnt16, dstl = _sc_prep(dst_p)
    hs = _tc_layer1(x, W1, cnt16)
    agg = _sc_agg(hs, src_p, dstl)
    hs = _tc_layer(agg, W2, b18, cnt16)
    agg = _sc_agg(hs, src_p, dstl)
    hs = _tc_layer(agg, W3, b28, cnt16)
    agg = _sc_agg(hs, src_p, dstl)
    out = _tc_final(agg, b38, cnt16, batch3, Wl1, bl18, Wl2p, bl28)
    return out[:, 0:1]


# speedup vs baseline: 11.6931x; 1.0633x over previous
import functools

import jax
import jax.numpy as jnp
from jax import lax
from jax.experimental import pallas as pl
from jax.experimental.pallas import tpu as pltpu
from jax.experimental.pallas import tpu_sc as plsc

N_NODES = 50000
HALF = 25000
ACC_ROWS = 25088
N_EDGES = 800000
HID = 64
NUM_GRAPHS = 64

NSUB = 16
NCORE = 2
CHUNK = 1024
CH_ROWS = CHUNK // 128
N_CHUNKS = 49
TILE_EDGES = CHUNK * N_CHUNKS
EPAD = TILE_EDGES * NSUB
EP_ROWS = EPAD // 128
TILE_ROWS = TILE_EDGES // 128

ROW_BLK = 2000
N_BLKS = N_NODES // ROW_BLK


def _mesh():
    return plsc.VectorSubcoreMesh(
        core_axis_name="c", subcore_axis_name="s",
        num_cores=NCORE, num_subcores=NSUB)


def _prep_body(dst_hbm, cnt16_hbm, dstl_hbm, dstb, dstlb, ones_v, zbuf,
               accd, sem, sem2):
    c = lax.axis_index("c")
    s = lax.axis_index("s")

    def fill(i, _):
        zbuf[i, :] = jnp.zeros((16,), jnp.float32)
        return _
    lax.fori_loop(0, 784, fill, None)

    def fill1(i, _):
        ones_v[i, :] = jnp.ones((16,), jnp.float32)
        return _
    lax.fori_loop(0, 128, fill1, None)

    pltpu.sync_copy(zbuf, accd.at[pl.ds(s * 1568, 784)])
    pltpu.sync_copy(zbuf, accd.at[pl.ds(s * 1568 + 784, 784)])
    plsc.subcore_barrier()

    def chunk(k, _):
        rb = s * TILE_ROWS + k * CH_ROWS
        pltpu.sync_copy(dst_hbm.at[pl.ds(rb, CH_ROWS)], dstb)
        for j in range(CH_ROWS):
            for v in range(8):
                d16 = dstb[j, pl.ds(v * 16, 16)]
                loc = d16 - c * HALF
                ok = (loc >= 0) & (loc < HALF)
                dstlb[j, pl.ds(v * 16, 16)] = jnp.where(ok, loc, HALF)
        pltpu.sync_copy(dstlb, dstl_hbm.at[c, pl.ds(rb, CH_ROWS)])

        def sub(j, _):
            cpa = pltpu.async_copy(ones_v, accd.at[dstlb.at[2 * j]], sem,
                                   add=True)
            cpb = pltpu.async_copy(ones_v, accd.at[dstlb.at[2 * j + 1]],
                                   sem2, add=True)
            cpa.wait()
            cpb.wait()
            return _
        lax.fori_loop(0, CH_ROWS // 2, sub, None)
        return _
    lax.fori_loop(0, N_CHUNKS, chunk, None)

    plsc.subcore_barrier()
    pltpu.sync_copy(accd.at[pl.ds(s * 1560, 1560)],
                    cnt16_hbm.at[pl.ds(c * HALF + s * 1560, 1560)])

    @pl.when(s == 0)
    def _():
        pltpu.sync_copy(accd.at[pl.ds(24960, 40)],
                        cnt16_hbm.at[pl.ds(c * HALF + 24960, 40)])


def _sc_prep(dst_p):
    fn = pl.kernel(
        _prep_body,
        out_type=(jax.ShapeDtypeStruct((N_NODES, 16), jnp.float32),
                  jax.ShapeDtypeStruct((NCORE, EP_ROWS, 128), jnp.int32)),
        mesh=_mesh(),
        compiler_params=pltpu.CompilerParams(use_tc_tiling_on_sc=False),
        scratch_types=[
            pltpu.VMEM((CH_ROWS, 128), jnp.int32),
            pltpu.VMEM((CH_ROWS, 128), jnp.int32),
            pltpu.VMEM((128, 16), jnp.float32),
            pltpu.VMEM((784, 16), jnp.float32),
            pltpu.VMEM_SHARED((ACC_ROWS, 16), jnp.float32),
            pltpu.SemaphoreType.DMA,
            pltpu.SemaphoreType.DMA,
        ],
    )
    return fn(dst_p)


def _agg_body(hs_hbm, src_hbm, dstl_hbm, agg_hbm, srcb, dstlb, rows_a,
              rows_b, acc, sem_a, sem_b):
    c = lax.axis_index("c")
    s = lax.axis_index("s")

    pltpu.sync_copy(hs_hbm.at[pl.ds(c * HALF + s * 1560, 1560)],
                    acc.at[pl.ds(s * 1560, 1560)])

    @pl.when(s == 0)
    def _():
        pltpu.sync_copy(hs_hbm.at[pl.ds(c * HALF + 24960, 40)],
                        acc.at[pl.ds(24960, 40)])

    plsc.subcore_barrier()

    def chunk(k, _):
        rb = s * TILE_ROWS + k * CH_ROWS
        pltpu.sync_copy(src_hbm.at[pl.ds(rb, CH_ROWS)], srcb)
        pltpu.sync_copy(dstl_hbm.at[c, pl.ds(rb, CH_ROWS)], dstlb)

        def sub(j, _):
            cpa = pltpu.async_copy(hs_hbm.at[srcb.at[2 * j]], rows_a,
                                   sem_a)
            cpb = pltpu.async_copy(hs_hbm.at[srcb.at[2 * j + 1]], rows_b,
                                   sem_b)
            cpa.wait()
            pltpu.sync_copy(rows_a, acc.at[dstlb.at[2 * j]], add=True)
            cpb.wait()
            pltpu.sync_copy(rows_b, acc.at[dstlb.at[2 * j + 1]], add=True)
            return _
        lax.fori_loop(0, CH_ROWS // 2, sub, None)
        return _
    lax.fori_loop(0, N_CHUNKS, chunk, None)

    plsc.subcore_barrier()
    pltpu.sync_copy(acc.at[pl.ds(s * 1560, 1560)],
                    agg_hbm.at[pl.ds(c * HALF + s * 1560, 1560)])

    @pl.when(s == 0)
    def _():
        pltpu.sync_copy(acc.at[pl.ds(24960, 40)],
                        agg_hbm.at[pl.ds(c * HALF + 24960, 40)])


def _sc_agg(hs, src_p, dstl):
    fn = pl.kernel(
        _agg_body,
        out_type=jax.ShapeDtypeStruct((N_NODES, HID), jnp.float32),
        mesh=_mesh(),
        compiler_params=pltpu.CompilerParams(use_tc_tiling_on_sc=False),
        scratch_types=[
            pltpu.VMEM((CH_ROWS, 128), jnp.int32),
            pltpu.VMEM((CH_ROWS, 128), jnp.int32),
            pltpu.VMEM((128, HID), jnp.float32),
            pltpu.VMEM((128, HID), jnp.float32),
            pltpu.VMEM_SHARED((ACC_ROWS, HID), jnp.float32),
            pltpu.SemaphoreType.DMA,
            pltpu.SemaphoreType.DMA,
        ],
    )
    return fn(hs, src_p, dstl)


def _bcast64(cnt16):
    dinv16 = lax.rsqrt(cnt16 + 1.0)
    sel = (lax.broadcasted_iota(jnp.int32, (16, HID), 0) == 0
           ).astype(jnp.float32)
    return lax.dot_general(dinv16, sel, (((1,), (0,)), ((), ())),
                           preferred_element_type=jnp.float32)


def _tc1_body(x_ref, w_ref, cnt_ref, out_ref):
    dinv = _bcast64(cnt_ref[...])
    h = lax.dot_general(x_ref[...], w_ref[...], (((1,), (1,)), ((), ())),
                        preferred_element_type=jnp.float32)
    out_ref[...] = h * dinv


def _tc_layer1(x, W1, cnt16):
    return pl.pallas_call(
        _tc1_body,
        grid=(N_BLKS,),
        in_specs=[
            pl.BlockSpec((ROW_BLK, 20), lambda i: (i, 0)),
            pl.BlockSpec((HID, 20), lambda i: (0, 0)),
            pl.BlockSpec((ROW_BLK, 16), lambda i: (i, 0)),
        ],
        out_specs=pl.BlockSpec((ROW_BLK, HID), lambda i: (i, 0)),
        out_shape=jax.ShapeDtypeStruct((N_NODES, HID), jnp.float32),
    )(x, W1, cnt16)


def _tc2_body(agg_ref, w_ref, b_ref, cnt_ref, out_ref):
    dinv = _bcast64(cnt_ref[...])
    h = jnp.maximum(agg_ref[...] * dinv + b_ref[0:1, :], 0.0)
    hw = lax.dot_general(h, w_ref[...], (((1,), (1,)), ((), ())),
                         preferred_element_type=jnp.float32)
    out_ref[...] = hw * dinv


def _tc_layer(agg, W, b8, cnt16):
    return pl.pallas_call(
        _tc2_body,
        grid=(N_BLKS,),
        in_specs=[
            pl.BlockSpec((ROW_BLK, HID), lambda i: (i, 0)),
            pl.BlockSpec((HID, HID), lambda i: (0, 0)),
            pl.BlockSpec((8, HID), lambda i: (0, 0)),
            pl.BlockSpec((ROW_BLK, 16), lambda i: (i, 0)),
        ],
        out_specs=pl.BlockSpec((ROW_BLK, HID), lambda i: (i, 0)),
        out_shape=jax.ShapeDtypeStruct((N_NODES, HID), jnp.float32),
    )(agg, W, b8, cnt16)


def _tc3_body(agg_ref, b_ref, cnt_ref, batch_ref, wl1_ref, bl1_ref,
              wl2_ref, bl2_ref, out_ref, pool_acc):
    i = pl.program_id(0)

    @pl.when(i == 0)
    def _():
        pool_acc[...] = jnp.zeros((NUM_GRAPHS, 128), jnp.float32)

    dinv = _bcast64(cnt_ref[...])
    h3 = jnp.maximum(agg_ref[...] * dinv + b_ref[0:1, :], 0.0)
    bids = batch_ref[0, 0, :]
    oh = (lax.broadcasted_iota(jnp.int32, (NUM_GRAPHS, ROW_BLK), 0)
          == bids[None, :]).astype(jnp.float32)
    psum = lax.dot_general(oh, h3, (((1,), (0,)), ((), ())),
                           preferred_element_type=jnp.float32)
    cnt64 = lax.dot_general(oh, jnp.ones((ROW_BLK, HID), jnp.float32),
                            (((1,), (0,)), ((), ())),
                            preferred_element_type=jnp.float32)
    pool_acc[:, 0:HID] += psum
    pool_acc[:, HID:128] += cnt64

    @pl.when(i == N_BLKS - 1)
    def _():
        pooled = pool_acc[:, 0:HID] / jnp.maximum(pool_acc[:, HID:128], 1.0)
        h = jnp.maximum(
            lax.dot_general(pooled, wl1_ref[...], (((1,), (1,)), ((), ())),
                            preferred_element_type=jnp.float32)
            + bl1_ref[0:1, :], 0.0)
        o = lax.dot_general(h, wl2_ref[...], (((1,), (1,)), ((), ())),
                            preferred_element_type=jnp.float32)
        out_ref[...] = o + bl2_ref[0:1, :]


def _tc_final(agg3, b38, cnt16, batch3, Wl1, bl18, Wl28, bl28):
    return pl.pallas_call(
        _tc3_body,
        grid=(N_BLKS,),
        in_specs=[
            pl.BlockSpec((ROW_BLK, HID), lambda i: (i, 0)),
            pl.BlockSpec((8, HID), lambda i: (0, 0)),
            pl.BlockSpec((ROW_BLK, 16), lambda i: (i, 0)),
            pl.BlockSpec((1, 1, ROW_BLK), lambda i: (i, 0, 0)),
            pl.BlockSpec((HID, HID), lambda i: (0, 0)),
            pl.BlockSpec((8, HID), lambda i: (0, 0)),
            pl.BlockSpec((128, HID), lambda i: (0, 0)),
            pl.BlockSpec((8, 128), lambda i: (0, 0)),
        ],
        out_specs=pl.BlockSpec((NUM_GRAPHS, 128), lambda i: (0, 0)),
        out_shape=jax.ShapeDtypeStruct((NUM_GRAPHS, 128), jnp.float32),
        scratch_shapes=[pltpu.VMEM((NUM_GRAPHS, 128), jnp.float32)],
    )(agg3, b38, cnt16, batch3, Wl1, bl18, Wl28, bl28)


@jax.jit
def kernel(x, edge_index, batch, W1, b1, W2, b2, W3, b3, Wl1, bl1, Wl2, bl2):
    src = edge_index[0].astype(jnp.int32)
    dst = edge_index[1].astype(jnp.int32)
    npad = EPAD - N_EDGES
    src_p = jnp.concatenate(
        [src, jnp.zeros((npad,), jnp.int32)]).reshape(EP_ROWS, 128)
    dst_p = jnp.concatenate(
        [dst, jnp.full((npad,), N_NODES, jnp.int32)]).reshape(EP_ROWS, 128)
    batch3 = batch.astype(jnp.int32).reshape(N_BLKS, 1, ROW_BLK)

    b18 = jnp.broadcast_to(b1[None, :], (8, HID))
    b28 = jnp.broadcast_to(b2[None, :], (8, HID))
    b38 = jnp.broadcast_to(b3[None, :], (8, HID))
    bl18 = jnp.broadcast_to(bl1[None, :], (8, HID))
    Wl2p = jnp.zeros((128, HID), jnp.float32).at[0:1, :].set(Wl2)
    bl28 = jnp.broadcast_to(bl2[None, :], (8, 128))

    cnt16, dstl = _sc_prep(dst_p)
    hs = _tc_layer1(x, W1, cnt16)
    agg = _sc_agg(hs, src_p, dstl)
    hs = _tc_layer(agg, W2, b18, cnt16)
    agg = _sc_agg(hs, src_p, dstl)
    hs = _tc_layer(agg, W3, b28, cnt16)
    agg = _sc_agg(hs, src_p, dstl)
    out = _tc_final(agg, b38, cnt16, batch3, Wl1, bl18, Wl2p, bl28)
    return out[:, 0:1]


# speedup vs baseline: 14.5223x; 1.2419x over previous
import functools

import jax
import jax.numpy as jnp
from jax import lax
from jax.experimental import pallas as pl
from jax.experimental.pallas import tpu as pltpu
from jax.experimental.pallas import tpu_sc as plsc

N_NODES = 50000
HALF = 25000
ACC_ROWS = 25088
N_EDGES = 800000
HID = 64
NUM_GRAPHS = 64

NSUB = 16
NCORE = 2
CHUNK = 1024
CH_ROWS = CHUNK // 128
N_CHUNKS = 49
TILE_EDGES = CHUNK * N_CHUNKS
EPAD = TILE_EDGES * NSUB
EP_ROWS = EPAD // 128
TILE_ROWS = TILE_EDGES // 128

CAP = 51200

ROW_BLK = 2000
N_BLKS = N_NODES // ROW_BLK


def _mesh():
    return plsc.VectorSubcoreMesh(
        core_axis_name="c", subcore_axis_name="s",
        num_cores=NCORE, num_subcores=NSUB)


def _prep_body(dst_hbm, src_hbm, cnt16_hbm, pk_hbm, counts_hbm,
               dstb, srcb, pk, outp, outd, ones_v, zbuf, accd, sem,
               sem2, cntv):
    c = lax.axis_index("c")
    s = lax.axis_index("s")

    def fill(i, _):
        zbuf[i, :] = jnp.zeros((16,), jnp.float32)
        return _
    lax.fori_loop(0, 112, fill, None)

    def fill1(i, _):
        ones_v[i, :] = jnp.ones((16,), jnp.float32)
        return _
    lax.fori_loop(0, 128, fill1, None)

    def zc(i, _):
        pltpu.sync_copy(zbuf, accd.at[pl.ds(s * 1568 + i * 112, 112)])
        return _
    lax.fori_loop(0, 14, zc, None)
    plsc.subcore_barrier()

    def chunk(k, off):
        rb = s * TILE_ROWS + k * CH_ROWS
        pltpu.sync_copy(dst_hbm.at[pl.ds(rb, CH_ROWS)], dstb)
        pltpu.sync_copy(src_hbm.at[pl.ds(rb, CH_ROWS)], srcb)
        for j in range(CH_ROWS):
            for v in range(8):
                d16 = dstb[j, pl.ds(v * 16, 16)]
                s16 = srcb[j, pl.ds(v * 16, 16)]
                loc = d16 - c * HALF
                ok = (loc >= 0) & (loc < HALF)
                packed = s16 | jnp.where(ok, loc, HALF) << 16
                plsc.store_compressed(pk.at[pl.ds(off, 16)], packed,
                                      mask=ok)
                off = off + jnp.max(plsc.all_reduce_population_count(ok))
        return off
    count = lax.fori_loop(0, N_CHUNKS, chunk, jnp.int32(0))

    target = ((count + 128 + 1023) // 1024) * 1024
    full = jnp.ones((16,), jnp.int32) > 0

    def padb(i, off):
        @pl.when(off < target)
        def _():
            plsc.store_compressed(pk.at[pl.ds(off, 16)],
                                  jnp.full((16,), HALF << 16, jnp.int32),
                                  mask=full)
        return jnp.where(off < target, off + 16, off)
    lax.fori_loop(0, 72, padb, count)

    def wb(w, _):
        base = w * 1024

        def cp(i, _):
            v = pk[pl.ds(base + i * 16, 16)]
            outp[pl.ds(i * 16, 16)] = v
            outd[pl.ds(i * 16, 16)] = lax.shift_right_logical(v, 16)
            return _
        lax.fori_loop(0, 64, cp, None)
        cps = pltpu.async_copy(
            outp, pk_hbm.at[c, pl.ds(s * CAP + base, 1024)], sem)
        cps.wait()

        def degb(p, _):
            cpa = pltpu.async_copy(
                ones_v, accd.at[outd.at[pl.ds(2 * p * 128, 128)]], sem,
                add=True)
            cpb = pltpu.async_copy(
                ones_v, accd.at[outd.at[pl.ds((2 * p + 1) * 128, 128)]],
                sem2, add=True)
            cpa.wait()
            cpb.wait()
            return _
        lax.fori_loop(0, 4, degb, None)
        return _
    lax.fori_loop(0, target // 1024, wb, None)

    cntv[pl.ds(0, 16)] = jnp.zeros((16,), jnp.int32) + target
    pltpu.sync_copy(cntv, counts_hbm.at[c, pl.ds(s * 16, 16)])

    plsc.subcore_barrier()
    pltpu.sync_copy(accd.at[pl.ds(s * 1560, 1560)],
                    cnt16_hbm.at[pl.ds(c * HALF + s * 1560, 1560)])

    @pl.when(s == 0)
    def _():
        pltpu.sync_copy(accd.at[pl.ds(24960, 40)],
                        cnt16_hbm.at[pl.ds(c * HALF + 24960, 40)])


def _sc_prep(dst_p, src_p):
    fn = pl.kernel(
        _prep_body,
        out_type=(jax.ShapeDtypeStruct((N_NODES, 16), jnp.float32),
                  jax.ShapeDtypeStruct((NCORE, NSUB * CAP), jnp.int32),
                  jax.ShapeDtypeStruct((NCORE, NSUB * 16), jnp.int32)),
        mesh=_mesh(),
        compiler_params=pltpu.CompilerParams(use_tc_tiling_on_sc=False,
                                             needs_layout_passes=False),
        scratch_types=[
            pltpu.VMEM((CH_ROWS, 128), jnp.int32),
            pltpu.VMEM((CH_ROWS, 128), jnp.int32),
            pltpu.VMEM((CAP,), jnp.int32),
            pltpu.VMEM((1024,), jnp.int32),
            pltpu.VMEM((1024,), jnp.int32),
            pltpu.VMEM((128, 16), jnp.float32),
            pltpu.VMEM((112, 16), jnp.float32),
            pltpu.VMEM_SHARED((ACC_ROWS, 16), jnp.float32),
            pltpu.SemaphoreType.DMA,
            pltpu.SemaphoreType.DMA,
            pltpu.VMEM((16,), jnp.int32),
        ],
    )
    return fn(dst_p, src_p)


def _agg_body(hs_hbm, pk_hbm, counts_hbm, agg_hbm, pkb, srcb_f,
              dstb_f, rows_a, rows_b, acc, sem_a, sem_b, cntv):
    c = lax.axis_index("c")
    s = lax.axis_index("s")

    pltpu.sync_copy(counts_hbm.at[c, pl.ds(s * 16, 16)], cntv)

    pltpu.sync_copy(hs_hbm.at[pl.ds(c * HALF + s * 1560, 1560)],
                    acc.at[pl.ds(s * 1560, 1560)])

    @pl.when(s == 0)
    def _():
        pltpu.sync_copy(hs_hbm.at[pl.ds(c * HALF + 24960, 40)],
                        acc.at[pl.ds(24960, 40)])
    plsc.subcore_barrier()

    def chunk(k, _):
        pltpu.sync_copy(pk_hbm.at[c, pl.ds(s * CAP + k * 1024, 1024)],
                        pkb)

        def unpack(i, _):
            v = pkb[pl.ds(i * 16, 16)]
            srcb_f[pl.ds(i * 16, 16)] = v & 65535
            dstb_f[pl.ds(i * 16, 16)] = lax.shift_right_logical(v, 16)
            return _
        lax.fori_loop(0, 64, unpack, None)

        def pair(p, _):
            cpa = pltpu.async_copy(
                hs_hbm.at[srcb_f.at[pl.ds(2 * p * 128, 128)]], rows_a,
                sem_a)
            cpb = pltpu.async_copy(
                hs_hbm.at[srcb_f.at[pl.ds((2 * p + 1) * 128, 128)]],
                rows_b, sem_b)
            cpa.wait()
            pltpu.sync_copy(rows_a, acc.at[dstb_f.at[pl.ds(2 * p * 128,
                                                           128)]],
                            add=True)
            cpb.wait()
            pltpu.sync_copy(
                rows_b, acc.at[dstb_f.at[pl.ds((2 * p + 1) * 128, 128)]],
                add=True)
            return _
        lax.fori_loop(0, 4, pair, None)
        return _
    n_chunks = jnp.max(cntv[pl.ds(0, 16)]) // 1024
    lax.fori_loop(0, n_chunks, chunk, None)

    plsc.subcore_barrier()
    pltpu.sync_copy(acc.at[pl.ds(s * 1560, 1560)],
                    agg_hbm.at[pl.ds(c * HALF + s * 1560, 1560)])

    @pl.when(s == 0)
    def _():
        pltpu.sync_copy(acc.at[pl.ds(24960, 40)],
                        agg_hbm.at[pl.ds(24960 + c * HALF, 40)])


def _sc_agg(hs, pk, counts):
    fn = pl.kernel(
        _agg_body,
        out_type=jax.ShapeDtypeStruct((N_NODES, HID), jnp.float32),
        mesh=_mesh(),
        compiler_params=pltpu.CompilerParams(use_tc_tiling_on_sc=False,
                                             needs_layout_passes=False),
        scratch_types=[
            pltpu.VMEM((1024,), jnp.int32),
            pltpu.VMEM((1024,), jnp.int32),
            pltpu.VMEM((1024,), jnp.int32),
            pltpu.VMEM((128, HID), jnp.float32),
            pltpu.VMEM((128, HID), jnp.float32),
            pltpu.VMEM_SHARED((ACC_ROWS, HID), jnp.float32),
            pltpu.SemaphoreType.DMA,
            pltpu.SemaphoreType.DMA,
            pltpu.VMEM((16,), jnp.int32),
        ],
    )
    return fn(hs, pk, counts)


def _bcast64(cnt16):
    dinv16 = lax.rsqrt(cnt16 + 1.0)
    sel = (lax.broadcasted_iota(jnp.int32, (16, HID), 0) == 0
           ).astype(jnp.float32)
    return lax.dot_general(dinv16, sel, (((1,), (0,)), ((), ())),
                           preferred_element_type=jnp.float32)


def _tc1_body(x_ref, w_ref, cnt_ref, out_ref):
    dinv = _bcast64(cnt_ref[...])
    h = lax.dot_general(x_ref[...], w_ref[...], (((1,), (1,)), ((), ())),
                        preferred_element_type=jnp.float32)
    out_ref[...] = h * dinv


def _tc_layer1(x, W1, cnt16):
    return pl.pallas_call(
        _tc1_body,
        grid=(N_BLKS,),
        in_specs=[
            pl.BlockSpec((ROW_BLK, 20), lambda i: (i, 0)),
            pl.BlockSpec((HID, 20), lambda i: (0, 0)),
            pl.BlockSpec((ROW_BLK, 16), lambda i: (i, 0)),
        ],
        out_specs=pl.BlockSpec((ROW_BLK, HID), lambda i: (i, 0)),
        out_shape=jax.ShapeDtypeStruct((N_NODES, HID), jnp.float32),
    )(x, W1, cnt16)


def _tc2_body(agg_ref, w_ref, b_ref, cnt_ref, out_ref):
    dinv = _bcast64(cnt_ref[...])
    h = jnp.maximum(agg_ref[...] * dinv + b_ref[0:1, :], 0.0)
    hw = lax.dot_general(h, w_ref[...], (((1,), (1,)), ((), ())),
                         preferred_element_type=jnp.float32)
    out_ref[...] = hw * dinv


def _tc_layer(agg, W, b8, cnt16):
    return pl.pallas_call(
        _tc2_body,
        grid=(N_BLKS,),
        in_specs=[
            pl.BlockSpec((ROW_BLK, HID), lambda i: (i, 0)),
            pl.BlockSpec((HID, HID), lambda i: (0, 0)),
            pl.BlockSpec((8, HID), lambda i: (0, 0)),
            pl.BlockSpec((ROW_BLK, 16), lambda i: (i, 0)),
        ],
        out_specs=pl.BlockSpec((ROW_BLK, HID), lambda i: (i, 0)),
        out_shape=jax.ShapeDtypeStruct((N_NODES, HID), jnp.float32),
    )(agg, W, b8, cnt16)


def _tc3_body(agg_ref, b_ref, cnt_ref, batch_ref, wl1_ref, bl1_ref,
              wl2_ref, bl2_ref, out_ref, pool_acc):
    i = pl.program_id(0)

    @pl.when(i == 0)
    def _():
        pool_acc[...] = jnp.zeros((NUM_GRAPHS, 128), jnp.float32)

    dinv = _bcast64(cnt_ref[...])
    h3 = jnp.maximum(agg_ref[...] * dinv + b_ref[0:1, :], 0.0)
    bids = batch_ref[0, 0, :]
    oh = (lax.broadcasted_iota(jnp.int32, (NUM_GRAPHS, ROW_BLK), 0)
          == bids[None, :]).astype(jnp.float32)
    psum = lax.dot_general(oh, h3, (((1,), (0,)), ((), ())),
                           preferred_element_type=jnp.float32)
    cnt64 = lax.dot_general(oh, jnp.ones((ROW_BLK, HID), jnp.float32),
                            (((1,), (0,)), ((), ())),
                            preferred_element_type=jnp.float32)
    pool_acc[:, 0:HID] += psum
    pool_acc[:, HID:128] += cnt64

    @pl.when(i == N_BLKS - 1)
    def _():
        pooled = pool_acc[:, 0:HID] / jnp.maximum(pool_acc[:, HID:128], 1.0)
        h = jnp.maximum(
            lax.dot_general(pooled, wl1_ref[...], (((1,), (1,)), ((), ())),
                            preferred_element_type=jnp.float32)
            + bl1_ref[0:1, :], 0.0)
        o = lax.dot_general(h, wl2_ref[...], (((1,), (1,)), ((), ())),
                            preferred_element_type=jnp.float32)
        out_ref[...] = o + bl2_ref[0:1, :]


def _tc_final(agg3, b38, cnt16, batch3, Wl1, bl18, Wl28, bl28):
    return pl.pallas_call(
        _tc3_body,
        grid=(N_BLKS,),
        in_specs=[
            pl.BlockSpec((ROW_BLK, HID), lambda i: (i, 0)),
            pl.BlockSpec((8, HID), lambda i: (0, 0)),
            pl.BlockSpec((ROW_BLK, 16), lambda i: (i, 0)),
            pl.BlockSpec((1, 1, ROW_BLK), lambda i: (i, 0, 0)),
            pl.BlockSpec((HID, HID), lambda i: (0, 0)),
            pl.BlockSpec((8, HID), lambda i: (0, 0)),
            pl.BlockSpec((128, HID), lambda i: (0, 0)),
            pl.BlockSpec((8, 128), lambda i: (0, 0)),
        ],
        out_specs=pl.BlockSpec((NUM_GRAPHS, 128), lambda i: (0, 0)),
        out_shape=jax.ShapeDtypeStruct((NUM_GRAPHS, 128), jnp.float32),
        scratch_shapes=[pltpu.VMEM((NUM_GRAPHS, 128), jnp.float32)],
    )(agg3, b38, cnt16, batch3, Wl1, bl18, Wl28, bl28)


@jax.jit
def kernel(x, edge_index, batch, W1, b1, W2, b2, W3, b3, Wl1, bl1, Wl2, bl2):
    src = edge_index[0].astype(jnp.int32)
    dst = edge_index[1].astype(jnp.int32)
    npad = EPAD - N_EDGES
    src_p = jnp.concatenate(
        [src, jnp.zeros((npad,), jnp.int32)]).reshape(EP_ROWS, 128)
    dst_p = jnp.concatenate(
        [dst, jnp.full((npad,), N_NODES, jnp.int32)]).reshape(EP_ROWS, 128)
    batch3 = batch.astype(jnp.int32).reshape(N_BLKS, 1, ROW_BLK)

    b18 = jnp.broadcast_to(b1[None, :], (8, HID))
    b28 = jnp.broadcast_to(b2[None, :], (8, HID))
    b38 = jnp.broadcast_to(b3[None, :], (8, HID))
    bl18 = jnp.broadcast_to(bl1[None, :], (8, HID))
    Wl2p = jnp.zeros((128, HID), jnp.float32).at[0:1, :].set(Wl2)
    bl28 = jnp.broadcast_to(bl2[None, :], (8, 128))

    cnt16, pk, counts = _sc_prep(dst_p, src_p)
    hs = _tc_layer1(x, W1, cnt16)
    agg = _sc_agg(hs, pk, counts)
    hs = _tc_layer(agg, W2, b18, cnt16)
    agg = _sc_agg(hs, pk, counts)
    hs = _tc_layer(agg, W3, b28, cnt16)
    agg = _sc_agg(hs, pk, counts)
    out = _tc_final(agg, b38, cnt16, batch3, Wl1, bl18, Wl2p, bl28)
    return out[:, 0:1]
